# Initial kernel scaffold; baseline (speedup 1.0000x reference)
import functools

import jax
import jax.numpy as jnp
from jax import lax
from jax.experimental import pallas as pl
from jax.experimental.pallas import tpu as pltpu
from jax.experimental.pallas import tpu_sc as plsc

N = 10000
E = 320000
D_IN = 128
D_MID = 256
D_OUT = 128

NP = 10240
NC = 2
NS = 16
CHUNK = 128

CA = -(-E // (NC * NS * CHUNK))
EA = NC * NS * CA * CHUNK
CB = -(-E // (NS * CHUNK))
EB = NS * CB * CHUNK

WA = 144


def _sc_agg_body(x_hbm, src_hbm, dst_hbm, z_hbm, out_hbm, idx_s, idx_d, rows, shared, sem, *, chunks, width):
    c = lax.axis_index("c")
    s = lax.axis_index("s")
    rps = NP // NS
    row0 = s * rps
    pltpu.sync_copy(z_hbm.at[pl.ds(row0, rps)], shared.at[pl.ds(row0, rps)])
    plsc.subcore_barrier()

    def chunk_body(i, carry):
        pltpu.sync_copy(src_hbm.at[c, s, i], idx_s)
        pltpu.sync_copy(dst_hbm.at[c, s, i], idx_d)
        pltpu.async_copy(x_hbm.at[idx_s], rows, sem).wait()
        pltpu.sync_copy(rows, shared.at[idx_d], add=True)
        return carry

    lax.fori_loop(0, chunks, chunk_body, 0)
    plsc.subcore_barrier()
    if width == WA:
        pltpu.sync_copy(shared.at[pl.ds(row0, rps), pl.ds(0, 128)],
                        out_hbm[0].at[c, pl.ds(row0, rps)])
        pltpu.sync_copy(shared.at[pl.ds(row0, rps), pl.ds(128, 8)],
                        out_hbm[1].at[c, pl.ds(row0, rps)])
    else:
        pltpu.sync_copy(shared.at[pl.ds(row0, rps)], out_hbm.at[c, pl.ds(row0, rps)])


def _make_sc_agg(chunks, width, two_outs):
    if two_outs:
        out_type = (jax.ShapeDtypeStruct((NC, NP, 128), jnp.float32),
                    jax.ShapeDtypeStruct((NC, NP, 8), jnp.float32))
    else:
        out_type = jax.ShapeDtypeStruct((NC, NP, width), jnp.float32)
    return pl.kernel(
        functools.partial(_sc_agg_body, chunks=chunks, width=width),
        out_type=out_type,
        mesh=plsc.VectorSubcoreMesh(core_axis_name="c", subcore_axis_name="s"),
        scratch_types=[
            pltpu.VMEM((CHUNK,), jnp.int32),
            pltpu.VMEM((CHUNK,), jnp.int32),
            pltpu.VMEM((CHUNK, width), jnp.float32),
            pltpu.VMEM_SHARED((NP, width), jnp.float32),
            pltpu.SemaphoreType.DMA,
        ],
        name=f"sage_sc_agg_{width}_{chunks}",
    )


_sc_agg_a = _make_sc_agg(CA, WA, True)
_sc_agg_b = _make_sc_agg(CB, 128, False)
_sc_agg_c = _make_sc_agg(CA, 128, False)


def _tc1_body(x_ref, p_ref, dg_ref, ws_ref, wn_ref, b_ref, o_ref):
    x = x_ref[:, :128]
    agg = p_ref[0] + p_ref[1]
    deg = dg_ref[0, :, 0] + dg_ref[1, :, 0]
    inv = 1.0 / jnp.maximum(deg, 1.0)
    h = (jnp.dot(x, ws_ref[...], preferred_element_type=jnp.float32)
         + jnp.dot(agg * inv[:, None], wn_ref[...], preferred_element_type=jnp.float32)
         + b_ref[...])
    o_ref[0] = jnp.maximum(h, 0.0)


def _tc2_body(x_ref, a_ref, dg_ref, ws1_ref, wn1_ref, b1_ref, ws2_ref, wn2_ref, b2_ref, s2_ref, p2_ref):
    xl, xh = x_ref[0], x_ref[1]
    al, ah = a_ref[0], a_ref[1]
    deg = dg_ref[0, :, 0] + dg_ref[1, :, 0]
    inv = (1.0 / jnp.maximum(deg, 1.0))[:, None]
    dot = lambda a, b: jnp.dot(a, b, preferred_element_type=jnp.float32)
    h = (dot(xl, ws1_ref[:128]) + dot(xh, ws1_ref[128:])
         + dot(al * inv, wn1_ref[:128]) + dot(ah * inv, wn1_ref[128:])
         + b1_ref[...])
    h = jnp.maximum(h, 0.0)
    s2_ref[...] = dot(h, ws2_ref[...]) + b2_ref[...]
    p2_ref[...] = dot(h, wn2_ref[...])


def _tc3_body(s2_ref, p_ref, dg_ref, o_ref):
    deg = dg_ref[0, :, 0] + dg_ref[1, :, 0]
    inv = (1.0 / jnp.maximum(deg, 1.0))[:, None]
    o_ref[...] = s2_ref[...] + (p_ref[0] + p_ref[1]) * inv


def kernel(features, edge_index, w_self0, w_neigh0, b0, w_self1, w_neigh1, b1, w_self2, w_neigh2, b2):
    f32 = jnp.float32
    src = edge_index[0].astype(jnp.int32)
    dst = edge_index[1].astype(jnp.int32)

    xaug = jnp.zeros((NP, WA), f32)
    xaug = lax.dynamic_update_slice(xaug, features, (0, 0))
    xaug = xaug.at[:N, 128].set(1.0)

    pad_a = EA - E
    src_a = jnp.concatenate([src, jnp.full((pad_a,), N, jnp.int32)]).reshape(NC, NS, CA, CHUNK)
    dst_a = jnp.concatenate([dst, jnp.full((pad_a,), N, jnp.int32)]).reshape(NC, NS, CA, CHUNK)
    pad_b = EB - E
    src_b1 = jnp.concatenate([src, jnp.full((pad_b,), N, jnp.int32)]).reshape(NS, CB, CHUNK)
    src_b = jnp.stack([src_b1, src_b1 + NP])
    dst_b1 = jnp.concatenate([dst, jnp.full((pad_b,), N, jnp.int32)]).reshape(NS, CB, CHUNK)
    dst_b = jnp.stack([dst_b1, dst_b1])

    z_a = jnp.zeros((NP, WA), f32)
    z_b = jnp.zeros((NP, 128), f32)

    part0, degp = _sc_agg_a(xaug, src_a, dst_a, z_a)

    b0r = b0.reshape(1, D_MID)
    x1h = pl.pallas_call(
        _tc1_body,
        grid=(NP // 512, 2),
        in_specs=[
            pl.BlockSpec((512, WA), lambda g, h: (g, 0)),
            pl.BlockSpec((2, 512, 128), lambda g, h: (0, g, 0)),
            pl.BlockSpec((2, 512, 8), lambda g, h: (0, g, 0)),
            pl.BlockSpec((128, 128), lambda g, h: (0, h)),
            pl.BlockSpec((128, 128), lambda g, h: (0, h)),
            pl.BlockSpec((1, 128), lambda g, h: (0, h)),
        ],
        out_specs=pl.BlockSpec((1, 512, 128), lambda g, h: (h, g, 0)),
        out_shape=jax.ShapeDtypeStruct((2, NP, 128), f32),
    )(xaug, part0, degp, w_self0, w_neigh0, b0r)

    agg1 = _sc_agg_b(x1h.reshape(2 * NP, 128), src_b, dst_b, z_b)

    b1r = b1.reshape(1, D_MID)
    b2r = b2.reshape(1, D_OUT)
    s2, p2 = pl.pallas_call(
        _tc2_body,
        grid=(NP // 512,),
        in_specs=[
            pl.BlockSpec((2, 512, 128), lambda g: (0, g, 0)),
            pl.BlockSpec((2, 512, 128), lambda g: (0, g, 0)),
            pl.BlockSpec((2, 512, 8), lambda g: (0, g, 0)),
            pl.BlockSpec((D_MID, D_MID), lambda g: (0, 0)),
            pl.BlockSpec((D_MID, D_MID), lambda g: (0, 0)),
            pl.BlockSpec((1, D_MID), lambda g: (0, 0)),
            pl.BlockSpec((D_MID, D_OUT), lambda g: (0, 0)),
            pl.BlockSpec((D_MID, D_OUT), lambda g: (0, 0)),
            pl.BlockSpec((1, D_OUT), lambda g: (0, 0)),
        ],
        out_specs=[
            pl.BlockSpec((512, 128), lambda g: (g, 0)),
            pl.BlockSpec((512, 128), lambda g: (g, 0)),
        ],
        out_shape=[
            jax.ShapeDtypeStruct((NP, 128), f32),
            jax.ShapeDtypeStruct((NP, 128), f32),
        ],
    )(x1h, agg1, degp, w_self1, w_neigh1, b1r, w_self2, w_neigh2, b2r)

    part2 = _sc_agg_c(p2, src_a, dst_a, z_b)

    out = pl.pallas_call(
        _tc3_body,
        grid=(N // 400,),
        in_specs=[
            pl.BlockSpec((400, 128), lambda g: (g, 0)),
            pl.BlockSpec((2, 400, 128), lambda g: (0, g, 0)),
            pl.BlockSpec((2, 400, 8), lambda g: (0, g, 0)),
        ],
        out_specs=pl.BlockSpec((400, 128), lambda g: (g, 0)),
        out_shape=jax.ShapeDtypeStruct((N, D_OUT), f32),
    )(s2, part2, degp)
    return out


# speedup vs baseline: 4.5984x; 4.5984x over previous
import functools

import jax
import jax.numpy as jnp
from jax import lax
from jax.experimental import pallas as pl
from jax.experimental.pallas import tpu as pltpu
from jax.experimental.pallas import tpu_sc as plsc

N = 10000
E = 320000
D_IN = 128
D_MID = 256
D_OUT = 128

NP = 10240
NC = 2
NS = 16
CHUNK = 128

CA = -(-E // (NC * NS * CHUNK))
EA = NC * NS * CA * CHUNK
CB = -(-E // (NS * CHUNK))
EB = NS * CB * CHUNK

W = 128


def _sc_agg_body(x_hbm, src_hbm, dst_hbm, z_hbm, *rest, chunks, with_deg):
    if with_deg:
        out_hbm, deg_hbm = rest[:2]
        idx_s, idx_d, rows, hist, shared, sem = rest[2:]
    else:
        out_hbm = rest[0]
        idx_s, idx_d, rows, shared, sem = rest[1:]
        hist = None
    c = lax.axis_index("c")
    s = lax.axis_index("s")
    rps = NP // NS
    row0 = s * rps
    pltpu.sync_copy(z_hbm.at[pl.ds(row0, rps)], shared.at[pl.ds(row0, rps)])
    plsc.subcore_barrier()
    if with_deg:
        zeros16 = jnp.zeros((16,), jnp.float32)

        def zbody(i, carry):
            hist[pl.ds(i * 16, 16)] = zeros16
            return carry

        lax.fori_loop(0, NP // 16, zbody, 0)
    ones16 = jnp.ones((16,), jnp.float32)

    def chunk_body(i, carry):
        pltpu.sync_copy(src_hbm.at[c, s, i], idx_s)
        pltpu.sync_copy(dst_hbm.at[c, s, i], idx_d)
        pltpu.async_copy(x_hbm.at[idx_s], rows, sem).wait()
        pltpu.sync_copy(rows, shared.at[idx_d], add=True)
        if with_deg:
            for j in range(CHUNK // 16):
                iv = idx_d[pl.ds(j * 16, 16)]
                plsc.addupdate_scatter(hist, [iv], ones16)
        return carry

    lax.fori_loop(0, chunks, chunk_body, 0)
    plsc.subcore_barrier()
    pltpu.sync_copy(shared.at[pl.ds(row0, rps)], out_hbm.at[c, pl.ds(row0, rps)])
    if with_deg:
        pltpu.sync_copy(hist, deg_hbm.at[c * NS + s])


def _make_sc_agg(chunks, with_deg):
    out_type = [jax.ShapeDtypeStruct((NC, NP, W), jnp.float32)]
    scratch = [
        pltpu.VMEM((CHUNK,), jnp.int32),
        pltpu.VMEM((CHUNK,), jnp.int32),
        pltpu.VMEM((CHUNK, W), jnp.float32),
        pltpu.VMEM_SHARED((NP, W), jnp.float32),
        pltpu.SemaphoreType.DMA,
    ]
    if with_deg:
        out_type.append(jax.ShapeDtypeStruct((NC * NS, NP), jnp.float32))
        scratch.insert(3, pltpu.VMEM((NP,), jnp.float32))
    return pl.kernel(
        functools.partial(_sc_agg_body, chunks=chunks, with_deg=with_deg),
        out_type=tuple(out_type),
        mesh=plsc.VectorSubcoreMesh(core_axis_name="c", subcore_axis_name="s"),
        scratch_types=scratch,
        compiler_params=pltpu.CompilerParams(needs_layout_passes=False),
        name=f"sage_sc_agg_{chunks}_{int(with_deg)}",
    )


_sc_agg_a = _make_sc_agg(CA, True)
_sc_agg_b = _make_sc_agg(CB, False)
_sc_agg_c = _make_sc_agg(CA, False)


def _tc1_body(x_ref, p_ref, dg_ref, ws_ref, wn_ref, b_ref, o_ref):
    x = x_ref[...]
    agg = p_ref[0] + p_ref[1]
    deg = jnp.sum(dg_ref[...], axis=0)
    inv = 1.0 / jnp.maximum(deg, 1.0)
    h = (jnp.dot(x, ws_ref[...], preferred_element_type=jnp.float32)
         + jnp.dot(agg * inv[:, None], wn_ref[...], preferred_element_type=jnp.float32)
         + b_ref[...])
    o_ref[0] = jnp.maximum(h, 0.0)


def _tc2_body(x_ref, a_ref, dg_ref, ws1_ref, wn1_ref, b1_ref, ws2_ref, wn2_ref, b2_ref, s2_ref, p2_ref):
    xl, xh = x_ref[0], x_ref[1]
    al, ah = a_ref[0], a_ref[1]
    deg = jnp.sum(dg_ref[...], axis=0)
    inv = (1.0 / jnp.maximum(deg, 1.0))[:, None]
    dot = lambda a, b: jnp.dot(a, b, preferred_element_type=jnp.float32)
    h = (dot(xl, ws1_ref[:128]) + dot(xh, ws1_ref[128:])
         + dot(al * inv, wn1_ref[:128]) + dot(ah * inv, wn1_ref[128:])
         + b1_ref[...])
    h = jnp.maximum(h, 0.0)
    s2_ref[...] = dot(h, ws2_ref[...]) + b2_ref[...]
    p2_ref[...] = dot(h, wn2_ref[...])


def _tc3_body(s2_ref, p_ref, dg_ref, o_ref):
    deg = jnp.sum(dg_ref[...], axis=0)
    inv = (1.0 / jnp.maximum(deg, 1.0))[:, None]
    o_ref[...] = s2_ref[...] + (p_ref[0] + p_ref[1]) * inv


def kernel(features, edge_index, w_self0, w_neigh0, b0, w_self1, w_neigh1, b1, w_self2, w_neigh2, b2):
    f32 = jnp.float32
    src = edge_index[0].astype(jnp.int32)
    dst = edge_index[1].astype(jnp.int32)

    xpad = jnp.zeros((NP, W), f32)
    xpad = lax.dynamic_update_slice(xpad, features, (0, 0))

    pad_a = EA - E
    src_a = jnp.concatenate([src, jnp.full((pad_a,), N, jnp.int32)]).reshape(NC, NS, CA, CHUNK)
    dst_a = jnp.concatenate([dst, jnp.full((pad_a,), N, jnp.int32)]).reshape(NC, NS, CA, CHUNK)
    pad_b = EB - E
    src_b1 = jnp.concatenate([src, jnp.full((pad_b,), N, jnp.int32)]).reshape(NS, CB, CHUNK)
    src_b = jnp.stack([src_b1, src_b1 + NP])
    dst_b1 = jnp.concatenate([dst, jnp.full((pad_b,), N, jnp.int32)]).reshape(NS, CB, CHUNK)
    dst_b = jnp.stack([dst_b1, dst_b1])

    z_b = jnp.zeros((NP, W), f32)

    part0, degp = _sc_agg_a(xpad, src_a, dst_a, z_b)

    b0r = b0.reshape(1, D_MID)
    x1h = pl.pallas_call(
        _tc1_body,
        grid=(NP // 512, 2),
        in_specs=[
            pl.BlockSpec((512, 128), lambda g, h: (g, 0)),
            pl.BlockSpec((2, 512, 128), lambda g, h: (0, g, 0)),
            pl.BlockSpec((NC * NS, 512), lambda g, h: (0, g)),
            pl.BlockSpec((128, 128), lambda g, h: (0, h)),
            pl.BlockSpec((128, 128), lambda g, h: (0, h)),
            pl.BlockSpec((1, 128), lambda g, h: (0, h)),
        ],
        out_specs=pl.BlockSpec((1, 512, 128), lambda g, h: (h, g, 0)),
        out_shape=jax.ShapeDtypeStruct((2, NP, 128), f32),
    )(xpad, part0, degp, w_self0, w_neigh0, b0r)

    agg1, = _sc_agg_b(x1h.reshape(2 * NP, 128), src_b, dst_b, z_b)

    b1r = b1.reshape(1, D_MID)
    b2r = b2.reshape(1, D_OUT)
    s2, p2 = pl.pallas_call(
        _tc2_body,
        grid=(NP // 512,),
        in_specs=[
            pl.BlockSpec((2, 512, 128), lambda g: (0, g, 0)),
            pl.BlockSpec((2, 512, 128), lambda g: (0, g, 0)),
            pl.BlockSpec((NC * NS, 512), lambda g: (0, g)),
            pl.BlockSpec((D_MID, D_MID), lambda g: (0, 0)),
            pl.BlockSpec((D_MID, D_MID), lambda g: (0, 0)),
            pl.BlockSpec((1, D_MID), lambda g: (0, 0)),
            pl.BlockSpec((D_MID, D_OUT), lambda g: (0, 0)),
            pl.BlockSpec((D_MID, D_OUT), lambda g: (0, 0)),
            pl.BlockSpec((1, D_OUT), lambda g: (0, 0)),
        ],
        out_specs=[
            pl.BlockSpec((512, 128), lambda g: (g, 0)),
            pl.BlockSpec((512, 128), lambda g: (g, 0)),
        ],
        out_shape=[
            jax.ShapeDtypeStruct((NP, 128), f32),
            jax.ShapeDtypeStruct((NP, 128), f32),
        ],
    )(x1h, agg1, degp, w_self1, w_neigh1, b1r, w_self2, w_neigh2, b2r)

    part2, = _sc_agg_c(p2, src_a, dst_a, z_b)

    out = pl.pallas_call(
        _tc3_body,
        grid=(NP // 512,),
        in_specs=[
            pl.BlockSpec((512, 128), lambda g: (g, 0)),
            pl.BlockSpec((2, 512, 128), lambda g: (0, g, 0)),
            pl.BlockSpec((NC * NS, 512), lambda g: (0, g)),
        ],
        out_specs=pl.BlockSpec((512, 128), lambda g: (g, 0)),
        out_shape=jax.ShapeDtypeStruct((N, D_OUT), f32),
    )(s2, part2, degp)
    return out
